# baseline (speedup 1.0000x reference)
import functools

import jax
import jax.numpy as jnp
from jax import lax
from jax.experimental import pallas as pl
from jax.experimental.pallas import tpu as pltpu
from jax.experimental.pallas import tpu_sc as plsc

_N = 10000
_E = 320000
_F = 128
_G = 256
_P = 480

_NC = 2
_NS = 16
_NW = _NC * _NS
_CH = 128
_BLK = 8
_NB = 10
_EC = _NB * _BLK
_EPW = _EC * _CH
_EP = _EPW * _NW
_NP = 10240
_RPT = _NP // _NS
_NPS = 12288
_RPW = _NPS // _NW
_SC = _RPW // _CH


@functools.lru_cache(maxsize=None)
def _edge_agg_kernel():
    mesh = plsc.VectorSubcoreMesh(core_axis_name="c", subcore_axis_name="s")

    @functools.partial(
        pl.kernel,
        out_type=jax.ShapeDtypeStruct((_NC, _NP, _F), jnp.float32),
        mesh=mesh,
        scratch_types=[
            pltpu.VMEM((_EC, _CH), jnp.int32),
            pltpu.VMEM((_EC, _CH), jnp.int32),
            pltpu.VMEM((_CH, _F), jnp.float32),
            pltpu.VMEM_SHARED((_NP, _F), jnp.float32),
        ],
    )
    def k(x_hbm, src_hbm, dst_hbm, zeros_hbm, out_hbm, src_v, dst_v, rows_v,
          acc_sh):
        c = lax.axis_index("c")
        s = lax.axis_index("s")
        wid = c * _NS + s
        pltpu.sync_copy(zeros_hbm.at[pl.ds(s * _RPT, _RPT)],
                        acc_sh.at[pl.ds(s * _RPT, _RPT)])
        pltpu.sync_copy(src_hbm.at[wid], src_v)
        pltpu.sync_copy(dst_hbm.at[wid], dst_v)
        plsc.subcore_barrier()

        def body(j, carry):
            pltpu.sync_copy(x_hbm.at[src_v.at[j]], rows_v)
            pltpu.sync_copy(rows_v, acc_sh.at[dst_v.at[j]], add=True)
            return carry

        lax.fori_loop(0, _EC, body, 0)
        plsc.subcore_barrier()
        pltpu.sync_copy(acc_sh.at[pl.ds(s * _RPT, _RPT)],
                        out_hbm.at[c, pl.ds(s * _RPT, _RPT)])

    return k


@functools.lru_cache(maxsize=None)
def _segsum_kernel():
    mesh = plsc.VectorSubcoreMesh(core_axis_name="c", subcore_axis_name="s")

    @functools.partial(
        pl.kernel,
        out_type=jax.ShapeDtypeStruct((_NC, _G, _F), jnp.float32),
        mesh=mesh,
        scratch_types=[
            pltpu.VMEM((_SC, _CH), jnp.int32),
            pltpu.VMEM((_CH, _F), jnp.float32),
            pltpu.VMEM_SHARED((_G, _F), jnp.float32),
        ],
    )
    def k(h_hbm, bat_hbm, zeros_hbm, out_hbm, bat_v, rows_v, acc_sh):
        c = lax.axis_index("c")
        s = lax.axis_index("s")
        wid = c * _NS + s
        gpt = _G // _NS
        pltpu.sync_copy(zeros_hbm.at[pl.ds(s * gpt, gpt)],
                        acc_sh.at[pl.ds(s * gpt, gpt)])
        pltpu.sync_copy(bat_hbm.at[wid], bat_v)
        plsc.subcore_barrier()

        def body(j, carry):
            pltpu.sync_copy(h_hbm.at[pl.ds(wid * _RPW + j * _CH, _CH)], rows_v)
            pltpu.sync_copy(rows_v, acc_sh.at[bat_v.at[j]], add=True)
            return carry

        lax.fori_loop(0, _SC, body, 0)
        plsc.subcore_barrier()
        pltpu.sync_copy(acc_sh.at[pl.ds(s * gpt, gpt)],
                        out_hbm.at[c, pl.ds(s * gpt, gpt)])

    return k


def _gin_dense_body(x_ref, agg_ref, w1_ref, b1_ref, g_ref, bb_ref, w2_ref,
                    b2_ref, o_ref):
    mask = (lax.broadcasted_iota(jnp.int32, (_NP, 1), 0) < _N).astype(jnp.float32)
    h = x_ref[...] + (agg_ref[0] + agg_ref[1])
    y = jnp.dot(h, w1_ref[...], preferred_element_type=jnp.float32) + b1_ref[...]
    m = jnp.sum(y * mask, axis=0, keepdims=True) * (1.0 / _N)
    dlt = (y - m) * mask
    v = jnp.sum(dlt * dlt, axis=0, keepdims=True) * (1.0 / _N)
    z = jnp.maximum((y - m) / jnp.sqrt(v + 1e-5) * g_ref[...] + bb_ref[...], 0.0)
    out = jnp.maximum(jnp.dot(z, w2_ref[...], preferred_element_type=jnp.float32)
                      + b2_ref[...], 0.0)
    o_ref[...] = out * mask


def _gin_dense(x_pad, agg, w1, b1, g, bb, w2, b2):
    return pl.pallas_call(
        _gin_dense_body,
        out_shape=jax.ShapeDtypeStruct((_NP, _F), jnp.float32),
    )(x_pad, agg, w1, b1.reshape(1, -1), g.reshape(1, -1), bb.reshape(1, -1),
      w2, b2.reshape(1, -1))


def _head_body(xg_ref, prot_ref, pw_ref, pb_ref, pg_ref, pbb_ref,
               w1_ref, b1_ref, g_ref, bb_ref, w2_ref, b2_ref, w3_ref, b3_ref,
               o_ref):
    xg = xg_ref[0] + xg_ref[1]
    t = jnp.dot(prot_ref[...], pw_ref[...],
                preferred_element_type=jnp.float32) + pb_ref[...]
    mu = jnp.mean(t, axis=1, keepdims=True)
    dt = t - mu
    var = jnp.mean(dt * dt, axis=1, keepdims=True)
    tn = (t - mu) / jnp.sqrt(var + 1e-5) * pg_ref[...] + pbb_ref[...]
    xp = 0.5 * tn * (1.0 + lax.erf(tn / 1.4142135623730951))
    xc = jnp.concatenate([xg, xp], axis=1)
    y = jnp.dot(xc, w1_ref[...], preferred_element_type=jnp.float32) + b1_ref[...]
    m = jnp.mean(y, axis=0, keepdims=True)
    dy = y - m
    v = jnp.mean(dy * dy, axis=0, keepdims=True)
    z = jnp.maximum((y - m) / jnp.sqrt(v + 1e-5) * g_ref[...] + bb_ref[...], 0.0)
    z = jnp.maximum(jnp.dot(z, w2_ref[...], preferred_element_type=jnp.float32)
                    + b2_ref[...], 0.0)
    o_ref[...] = jnp.dot(z, w3_ref[...], preferred_element_type=jnp.float32) \
        + b3_ref[...]


def _head(xg, prot, pw, pb, pg, pbb, w1, b1, g, bb, w2, b2, w3, b3):
    return pl.pallas_call(
        _head_body,
        out_shape=jax.ShapeDtypeStruct((_G, 1), jnp.float32),
    )(xg, prot, pw, pb.reshape(1, -1), pg.reshape(1, -1), pbb.reshape(1, -1),
      w1, b1.reshape(1, -1), g.reshape(1, -1), bb.reshape(1, -1),
      w2, b2.reshape(1, -1), w3, b3.reshape(1, -1))


def kernel(x, edge_index, batch, protein_emb,
           c1_w1, c1_b1, c1_g, c1_bb, c1_w2, c1_b2,
           c2_w1, c2_b1, c2_g, c2_bb, c2_w2, c2_b2,
           c3_w1, c3_b1, c3_g, c3_bb, c3_w2, c3_b2,
           prot_w, prot_b, prot_g, prot_bb,
           pred_w1, pred_b1, pred_g, pred_bb, pred_w2, pred_b2, pred_w3, pred_b3):
    f32 = jnp.float32
    x_pad = jnp.zeros((_NP, _F), f32).at[:_N].set(x)
    src = jnp.concatenate(
        [edge_index[0], jnp.full((_EP - _E,), _N, jnp.int32)]
    ).reshape(_NW, _EC, _CH)
    dst = jnp.concatenate(
        [edge_index[1], jnp.zeros((_EP - _E,), jnp.int32)]
    ).reshape(_NW, _EC, _CH)
    bat = jnp.concatenate(
        [batch, jnp.zeros((_NPS - _N,), jnp.int32)]).reshape(_NW, _SC, _CH)
    zeros_n = jnp.zeros((_NP, _F), f32)
    zeros_g = jnp.zeros((_G, _F), f32)

    edge_agg = _edge_agg_kernel()
    h = x_pad
    for (w1, b1, g, bb, w2, b2) in (
            (c1_w1, c1_b1, c1_g, c1_bb, c1_w2, c1_b2),
            (c2_w1, c2_b1, c2_g, c2_bb, c2_w2, c2_b2),
            (c3_w1, c3_b1, c3_g, c3_bb, c3_w2, c3_b2)):
        agg = edge_agg(h, src, dst, zeros_n)
        h = _gin_dense(h, agg, w1, b1, g, bb, w2, b2)

    h_seg = jnp.zeros((_NPS, _F), f32).at[:_NP].set(h)
    xg = _segsum_kernel()(h_seg, bat, zeros_g)
    prot = protein_emb.reshape(_G, _P)
    z = _head(xg, prot, prot_w, prot_b, prot_g, prot_bb,
              pred_w1, pred_b1, pred_g, pred_bb,
              pred_w2, pred_b2, pred_w3, pred_b3)
    return z.reshape(-1)


# speedup vs baseline: 3.8196x; 3.8196x over previous
import functools

import jax
import jax.numpy as jnp
from jax import lax
from jax.experimental import pallas as pl
from jax.experimental.pallas import tpu as pltpu
from jax.experimental.pallas import tpu_sc as plsc

_N = 10000
_E = 320000
_F = 128
_G = 256
_P = 480

_NC = 2
_NS = 16
_NW = _NC * _NS
_CH = 128
_BLK = 8
_NB = 10
_EC = _NB * _BLK
_EPW = _EC * _CH
_EP = _EPW * _NW
_ISL = 3 * _BLK
_NP = 10240
_RPT = _NP // _NS
_NPS = 12288
_RPW = _NPS // _NW
_SC = _RPW // _CH


@functools.lru_cache(maxsize=None)
def _edge_agg_kernel():
    mesh = plsc.VectorSubcoreMesh(core_axis_name="c", subcore_axis_name="s")

    @functools.partial(
        pl.kernel,
        out_type=jax.ShapeDtypeStruct((_NC, _NP, _F), jnp.float32),
        mesh=mesh,
        scratch_types=[
            pltpu.VMEM((_ISL, _CH), jnp.int32),
            pltpu.VMEM((_ISL, _CH), jnp.int32),
            pltpu.VMEM((2, _CH, _F), jnp.float32),
            pltpu.VMEM_SHARED((_NP, _F), jnp.float32),
            pltpu.SemaphoreType.DMA,
            pltpu.SemaphoreType.DMA,
        ],
    )
    def k(x_hbm, src_hbm, dst_hbm, zeros_hbm, out_hbm, si_v, di_v, r_v, acc_sh,
          g0, g1):
        c = lax.axis_index("c")
        s = lax.axis_index("s")
        wid = c * _NS + s
        sems = (g0, g1)
        pltpu.sync_copy(src_hbm.at[wid, pl.ds(0, 2 * _BLK)],
                        si_v.at[pl.ds(0, 2 * _BLK)])
        pltpu.sync_copy(dst_hbm.at[wid, pl.ds(0, 2 * _BLK)],
                        di_v.at[pl.ds(0, 2 * _BLK)])
        pltpu.async_copy(x_hbm.at[si_v.at[0]], r_v.at[0], g0)
        pltpu.async_copy(x_hbm.at[si_v.at[1]], r_v.at[1], g1)
        pltpu.sync_copy(zeros_hbm.at[pl.ds(s * _RPT, _RPT)],
                        acc_sh.at[pl.ds(s * _RPT, _RPT)])
        plsc.subcore_barrier()

        def body(i, carry):
            t0 = 2 * i

            @pl.when(jnp.logical_and(lax.rem(t0, _BLK) == 0, t0 + _BLK < _EC))
            def _():
                hb = pl.multiple_of(t0 + _BLK, _BLK)
                sl = pl.multiple_of(lax.rem(t0 + _BLK, _ISL), _BLK)
                pltpu.sync_copy(src_hbm.at[wid, pl.ds(hb, _BLK)],
                                si_v.at[pl.ds(sl, _BLK)])
                pltpu.sync_copy(dst_hbm.at[wid, pl.ds(hb, _BLK)],
                                di_v.at[pl.ds(sl, _BLK)])

            for q in range(2):
                t = t0 + q
                pltpu.make_async_copy(x_hbm.at[si_v.at[lax.rem(t, _ISL)]],
                                      r_v.at[q], sems[q]).wait()
                pltpu.sync_copy(r_v.at[q], acc_sh.at[di_v.at[lax.rem(t, _ISL)]],
                                add=True)

                @pl.when(t + 2 < _EC)
                def _():
                    pltpu.async_copy(x_hbm.at[si_v.at[lax.rem(t + 2, _ISL)]],
                                     r_v.at[q], sems[q])

            return carry

        lax.fori_loop(0, _EC // 2, body, 0)
        plsc.subcore_barrier()
        pltpu.sync_copy(acc_sh.at[pl.ds(s * _RPT, _RPT)],
                        out_hbm.at[c, pl.ds(s * _RPT, _RPT)])

    return k


@functools.lru_cache(maxsize=None)
def _segsum_kernel():
    mesh = plsc.VectorSubcoreMesh(core_axis_name="c", subcore_axis_name="s")

    @functools.partial(
        pl.kernel,
        out_type=jax.ShapeDtypeStruct((_NC, _G, _F), jnp.float32),
        mesh=mesh,
        scratch_types=[
            pltpu.VMEM((_SC, _CH), jnp.int32),
            pltpu.VMEM((_CH, _F), jnp.float32),
            pltpu.VMEM_SHARED((_G, _F), jnp.float32),
        ],
    )
    def k(h_hbm, bat_hbm, zeros_hbm, out_hbm, bat_v, rows_v, acc_sh):
        c = lax.axis_index("c")
        s = lax.axis_index("s")
        wid = c * _NS + s
        gpt = _G // _NS
        pltpu.sync_copy(zeros_hbm.at[pl.ds(s * gpt, gpt)],
                        acc_sh.at[pl.ds(s * gpt, gpt)])
        pltpu.sync_copy(bat_hbm.at[wid], bat_v)
        plsc.subcore_barrier()

        def body(j, carry):
            pltpu.sync_copy(h_hbm.at[pl.ds(wid * _RPW + j * _CH, _CH)], rows_v)
            pltpu.sync_copy(rows_v, acc_sh.at[bat_v.at[j]], add=True)
            return carry

        lax.fori_loop(0, _SC, body, 0)
        plsc.subcore_barrier()
        pltpu.sync_copy(acc_sh.at[pl.ds(s * gpt, gpt)],
                        out_hbm.at[c, pl.ds(s * gpt, gpt)])

    return k


def _gin_dense_body(x_ref, agg_ref, w1_ref, b1_ref, g_ref, bb_ref, w2_ref,
                    b2_ref, o_ref):
    mask = (lax.broadcasted_iota(jnp.int32, (_NP, 1), 0) < _N).astype(jnp.float32)
    h = x_ref[...] + (agg_ref[0] + agg_ref[1])
    y = jnp.dot(h, w1_ref[...], preferred_element_type=jnp.float32) + b1_ref[...]
    m = jnp.sum(y * mask, axis=0, keepdims=True) * (1.0 / _N)
    dlt = (y - m) * mask
    v = jnp.sum(dlt * dlt, axis=0, keepdims=True) * (1.0 / _N)
    z = jnp.maximum((y - m) / jnp.sqrt(v + 1e-5) * g_ref[...] + bb_ref[...], 0.0)
    out = jnp.maximum(jnp.dot(z, w2_ref[...], preferred_element_type=jnp.float32)
                      + b2_ref[...], 0.0)
    o_ref[...] = out * mask


def _gin_dense(x_pad, agg, w1, b1, g, bb, w2, b2):
    return pl.pallas_call(
        _gin_dense_body,
        out_shape=jax.ShapeDtypeStruct((_NP, _F), jnp.float32),
    )(x_pad, agg, w1, b1.reshape(1, -1), g.reshape(1, -1), bb.reshape(1, -1),
      w2, b2.reshape(1, -1))


def _head_body(xg_ref, prot_ref, pw_ref, pb_ref, pg_ref, pbb_ref,
               w1_ref, b1_ref, g_ref, bb_ref, w2_ref, b2_ref, w3_ref, b3_ref,
               o_ref):
    xg = xg_ref[0] + xg_ref[1]
    t = jnp.dot(prot_ref[...], pw_ref[...],
                preferred_element_type=jnp.float32) + pb_ref[...]
    mu = jnp.mean(t, axis=1, keepdims=True)
    dt = t - mu
    var = jnp.mean(dt * dt, axis=1, keepdims=True)
    tn = (t - mu) / jnp.sqrt(var + 1e-5) * pg_ref[...] + pbb_ref[...]
    xp = 0.5 * tn * (1.0 + lax.erf(tn / 1.4142135623730951))
    xc = jnp.concatenate([xg, xp], axis=1)
    y = jnp.dot(xc, w1_ref[...], preferred_element_type=jnp.float32) + b1_ref[...]
    m = jnp.mean(y, axis=0, keepdims=True)
    dy = y - m
    v = jnp.mean(dy * dy, axis=0, keepdims=True)
    z = jnp.maximum((y - m) / jnp.sqrt(v + 1e-5) * g_ref[...] + bb_ref[...], 0.0)
    z = jnp.maximum(jnp.dot(z, w2_ref[...], preferred_element_type=jnp.float32)
                    + b2_ref[...], 0.0)
    o_ref[...] = jnp.dot(z, w3_ref[...], preferred_element_type=jnp.float32) \
        + b3_ref[...]


def _head(xg, prot, pw, pb, pg, pbb, w1, b1, g, bb, w2, b2, w3, b3):
    return pl.pallas_call(
        _head_body,
        out_shape=jax.ShapeDtypeStruct((_G, 1), jnp.float32),
    )(xg, prot, pw, pb.reshape(1, -1), pg.reshape(1, -1), pbb.reshape(1, -1),
      w1, b1.reshape(1, -1), g.reshape(1, -1), bb.reshape(1, -1),
      w2, b2.reshape(1, -1), w3, b3.reshape(1, -1))


def kernel(x, edge_index, batch, protein_emb,
           c1_w1, c1_b1, c1_g, c1_bb, c1_w2, c1_b2,
           c2_w1, c2_b1, c2_g, c2_bb, c2_w2, c2_b2,
           c3_w1, c3_b1, c3_g, c3_bb, c3_w2, c3_b2,
           prot_w, prot_b, prot_g, prot_bb,
           pred_w1, pred_b1, pred_g, pred_bb, pred_w2, pred_b2, pred_w3, pred_b3):
    f32 = jnp.float32
    x_pad = jnp.zeros((_NP, _F), f32).at[:_N].set(x)
    pad_rows = _N + jnp.arange(_EP - _E, dtype=jnp.int32) % (_NP - _N)
    src = jnp.concatenate([edge_index[0], pad_rows]).reshape(_NW, _EC, _CH)
    dst = jnp.concatenate([edge_index[1], pad_rows]).reshape(_NW, _EC, _CH)
    bat = jnp.concatenate(
        [batch, jnp.arange(_NPS - _N, dtype=jnp.int32) % _G]
    ).reshape(_NW, _SC, _CH)
    zeros_n = jnp.zeros((_NP, _F), f32)
    zeros_g = jnp.zeros((_G, _F), f32)

    edge_agg = _edge_agg_kernel()
    h = x_pad
    for (w1, b1, g, bb, w2, b2) in (
            (c1_w1, c1_b1, c1_g, c1_bb, c1_w2, c1_b2),
            (c2_w1, c2_b1, c2_g, c2_bb, c2_w2, c2_b2),
            (c3_w1, c3_b1, c3_g, c3_bb, c3_w2, c3_b2)):
        agg = edge_agg(h, src, dst, zeros_n)
        h = _gin_dense(h, agg, w1, b1, g, bb, w2, b2)

    h_seg = jnp.zeros((_NPS, _F), f32).at[:_NP].set(h)
    xg = _segsum_kernel()(h_seg, bat, zeros_g)
    prot = protein_emb.reshape(_G, _P)
    z = _head(xg, prot, prot_w, prot_b, prot_g, prot_bb,
              pred_w1, pred_b1, pred_g, pred_bb,
              pred_w2, pred_b2, pred_w3, pred_b3)
    return z.reshape(-1)
